# Initial kernel scaffold; baseline (speedup 1.0000x reference)
import jax
import jax.numpy as jnp
from jax.experimental import pallas as pl


def kernel(atype, emb_table):
    raise NotImplementedError("write your pallas kernel here")


# speedup vs baseline: 5.4648x; 5.4648x over previous
import functools

import jax
import jax.numpy as jnp
from jax import lax
from jax.experimental import pallas as pl
from jax.experimental.pallas import tpu as pltpu
from jax.experimental.pallas import tpu_sc as plsc

D = 128
B = 16384 * 200
NW = 32
ROWS_PER_W = B // NW
G = 128
NGROUPS = ROWS_PER_W // G

_mesh = plsc.VectorSubcoreMesh(core_axis_name="c", subcore_axis_name="s")


@functools.partial(
    pl.kernel,
    mesh=_mesh,
    out_type=jax.ShapeDtypeStruct((B, D), jnp.float32),
    scratch_types=[
        pltpu.VMEM((G,), jnp.int32),
        pltpu.VMEM((G, D), jnp.float32),
        pltpu.SemaphoreType.DMA,
    ],
)
def _emb_gather(idx_hbm, table_hbm, out_hbm, idx_v, rows_v, sem):
    wid = lax.axis_index("s") * 2 + lax.axis_index("c")
    w_base = wid * ROWS_PER_W

    def body(g, carry):
        base = w_base + g * G
        pltpu.sync_copy(idx_hbm.at[pl.ds(base, G)], idx_v)
        pltpu.async_copy(table_hbm.at[idx_v], rows_v, sem).wait()
        pltpu.sync_copy(rows_v, out_hbm.at[pl.ds(base, G)])
        return carry

    lax.fori_loop(0, NGROUPS, body, 0)


def kernel(atype, emb_table):
    shape = atype.shape
    flat = atype.reshape(-1).astype(jnp.int32)
    out = _emb_gather(flat, emb_table)
    return out.reshape(*shape, D)
